# Initial kernel scaffold; baseline (speedup 1.0000x reference)
import jax
import jax.numpy as jnp
from jax.experimental import pallas as pl


def kernel(x, table):
    raise NotImplementedError("write your pallas kernel here")


# speedup vs baseline: 1.8448x; 1.8448x over previous
import functools

import jax
import jax.numpy as jnp
from jax import lax
from jax.experimental import pallas as pl
from jax.experimental.pallas import tpu as pltpu
from jax.experimental.pallas import tpu_sc as plsc

_NC = 2
_NS = 16
_NW = _NC * _NS
_IW = 128
_K = 8


def _build(B, V, D):
    b_per_w = B // _NW
    chunk = _K * _IW
    nchunk = b_per_w // chunk
    mesh = plsc.VectorSubcoreMesh(core_axis_name="c", subcore_axis_name="s")

    @functools.partial(
        pl.kernel,
        mesh=mesh,
        out_type=jax.ShapeDtypeStruct((B, D), jnp.float32),
        scratch_types=[
            pltpu.VMEM((_K, _IW), jnp.int32),
            pltpu.VMEM((chunk, D), jnp.float32),
            pltpu.SemaphoreType.DMA,
        ],
        compiler_params=pltpu.CompilerParams(use_tc_tiling_on_sc=False),
    )
    def k(idx_hbm, table_hbm, out_hbm, idx_v, rows_v, sem):
        wid = lax.axis_index("s") * _NC + lax.axis_index("c")

        def body(g, _):
            rowbase = (wid * nchunk + g) * _K
            pltpu.sync_copy(idx_hbm.at[pl.ds(rowbase, _K)], idx_v)
            copies = [
                pltpu.async_copy(
                    table_hbm.at[idx_v.at[j]],
                    rows_v.at[pl.ds(j * _IW, _IW)],
                    sem,
                )
                for j in range(_K)
            ]
            for c in copies:
                c.wait()
            pltpu.sync_copy(rows_v, out_hbm.at[pl.ds(rowbase * _IW, chunk)])
            return _

        lax.fori_loop(0, nchunk, body, None)

    return k


def kernel(x, table):
    B0, B1 = x.shape
    V, D = table.shape
    B = B0 * B1
    idx2d = x.reshape(B // _IW, _IW).astype(jnp.int32)
    out = _build(B, V, D)(idx2d, table)
    return out.reshape(B0, B1, D)


# speedup vs baseline: 1.8532x; 1.0046x over previous
import functools

import jax
import jax.numpy as jnp
from jax import lax
from jax.experimental import pallas as pl
from jax.experimental.pallas import tpu as pltpu
from jax.experimental.pallas import tpu_sc as plsc

_NC = 2
_NS = 16
_NW = _NC * _NS
_IW = 128
_K = 4
_NBUF = 2


def _build(B, V, D):
    b_per_w = B // _NW
    chunk = _K * _IW
    nchunk = b_per_w // chunk
    mesh = plsc.VectorSubcoreMesh(core_axis_name="c", subcore_axis_name="s")

    @functools.partial(
        pl.kernel,
        mesh=mesh,
        out_type=jax.ShapeDtypeStruct((B, D), jnp.float32),
        scratch_types=[
            pltpu.VMEM((_K, _IW), jnp.int32),
            pltpu.VMEM((_K, _IW), jnp.int32),
            pltpu.VMEM((chunk, D), jnp.float32),
            pltpu.VMEM((chunk, D), jnp.float32),
            pltpu.SemaphoreType.DMA,
            pltpu.SemaphoreType.DMA,
            pltpu.SemaphoreType.DMA,
            pltpu.SemaphoreType.DMA,
        ],
        compiler_params=pltpu.CompilerParams(use_tc_tiling_on_sc=False),
    )
    def k(idx_hbm, table_hbm, out_hbm, idx0, idx1, rows0, rows1, g0, g1, w0, w1):
        idx_b = (idx0, idx1)
        rows_b = (rows0, rows1)
        gs = (g0, g1)
        ws = (w0, w1)
        wid = lax.axis_index("s") * _NC + lax.axis_index("c")
        cbase = wid * nchunk

        def fire(g, b):
            rowbase = (cbase + g) * _K
            pltpu.sync_copy(idx_hbm.at[pl.ds(rowbase, _K)], idx_b[b])
            for j in range(_K):
                pltpu.async_copy(
                    table_hbm.at[idx_b[b].at[j]],
                    rows_b[b].at[pl.ds(j * _IW, _IW)],
                    gs[b],
                )

        def gather_wait(b):
            pltpu.make_async_copy(out_hbm.at[pl.ds(0, chunk)], rows_b[b], gs[b]).wait()

        def writeback(g, b):
            pltpu.async_copy(
                rows_b[b], out_hbm.at[pl.ds((cbase + g) * chunk, chunk)], ws[b]
            )

        def wb_wait(b):
            pltpu.make_async_copy(rows_b[b], out_hbm.at[pl.ds(0, chunk)], ws[b]).wait()

        def body(i, carry):
            for b in range(_NBUF):
                g = _NBUF * i + b

                @pl.when(g >= _NBUF)
                def _():
                    wb_wait(b)

                fire(g, b)

                @pl.when(g >= 1)
                def _():
                    gather_wait(1 - b)
                    writeback(g - 1, 1 - b)

            return carry

        lax.fori_loop(0, nchunk // _NBUF, body, None)
        gather_wait((nchunk - 1) % _NBUF)
        writeback(nchunk - 1, (nchunk - 1) % _NBUF)
        for b in range(_NBUF):
            wb_wait(b)

    return k


def kernel(x, table):
    B0, B1 = x.shape
    V, D = table.shape
    B = B0 * B1
    idx2d = x.reshape(B // _IW, _IW).astype(jnp.int32)
    out = _build(B, V, D)(idx2d, table)
    return out.reshape(B0, B1, D)
